# baseline (speedup 1.0000x reference)
import jax
import jax.numpy as jnp
from jax import lax
from jax.experimental import pallas as pl
from jax.experimental.pallas import tpu as pltpu, tpu_sc as plsc

NC = 2
NS = 16
LN = 16
NW = NC * NS

P = 16384
DIM = 100
GR = 7
BPW = P // NW
SLOTS = 2 * BPW
GROUPS = SLOTS // LN
CGROUPS = BPW // LN
NIDX = SLOTS * GR
KCH = NIDX // 128


def _sc_body(table_hbm, rel_hbm, reln_hbm, pr_hbm, prp_hbm, out_hbm,
             e_v, o_v, idx_v, rows_v, pr_v, prp_v, tg_v, tmp_v, part_v,
             sem):
    wid = lax.axis_index("s") * NC + lax.axis_index("c")
    base = wid * BPW

    pltpu.sync_copy(rel_hbm.at[pl.ds(base, BPW)], e_v.at[pl.ds(0, BPW)])
    pltpu.sync_copy(reln_hbm.at[pl.ds(base, BPW)], e_v.at[pl.ds(BPW, BPW)])
    pltpu.sync_copy(pr_hbm.at[pl.ds(base, BPW)], pr_v)
    pltpu.sync_copy(prp_hbm.at[pl.ds(base, BPW)], prp_v)
    pltpu.sync_copy(table_hbm.at[pl.ds(0, 20)], tg_v)

    iota = lax.broadcasted_iota(jnp.int32, (LN,), 0)

    def build(g, pb):
        e16 = e_v[pl.ds(g * LN, LN)]
        g0 = (e16 << 2) + (e16 << 1) + (e16 >> 2)
        o_v[pl.ds(g * LN, LN)] = (e16 & 3) << 2
        for j in range(GR):
            p = pb + j
            plsc.store_scatter(idx_v, [p >> 7, p & 127], g0 + j)
        return pb + GR * LN

    lax.fori_loop(0, GROUPS, build, GR * iota)

    copies = []
    for k in range(KCH):
        copies.append(pltpu.async_copy(
            table_hbm.at[idx_v.at[k]],
            rows_v.at[pl.ds(k * 128, 128)], sem))

    zero16 = jnp.zeros((LN,), jnp.int32)
    for c in range(GR):
        dd = jnp.minimum(c * LN + iota, DIM - 1)
        t = jnp.zeros((LN,), jnp.float32)
        for j in range(3):
            w = j * DIM + dd
            t = t + plsc.load_gather(tg_v, [w >> 4, w & 15])
        tmp_v[c] = t

    for c in copies:
        c.wait()

    zeros = jnp.zeros((LN,), jnp.float32)

    def group(g, partial):
        op16 = o_v[pl.ds(g * LN, LN)]
        on16 = o_v[pl.ds(BPW + g * LN, LN)]
        wbase = (GR * LN * LN) * g + (GR * LN) * iota
        up0 = wbase + op16
        un0 = wbase + (GR * LN) * BPW + on16

        def dbody(d, carry):
            ap, an, up, un = carry
            t = plsc.load_gather(tmp_v, [jnp.full((LN,), d >> 4, jnp.int32),
                                         jnp.full((LN,), d & 15, jnp.int32)])
            vp = plsc.load_gather(rows_v, [up >> 4, up & 15])
            vn = plsc.load_gather(rows_v, [un >> 4, un & 15])
            return (ap + jnp.abs(vp - t), an + jnp.abs(vn - t),
                    up + 1, un + 1)

        ap, an, _, _ = lax.fori_loop(0, DIM, dbody, (zeros, zeros, up0, un0))
        xs = pr_v[pl.ds(g * LN, LN)] * (0.99 * prp_v[pl.ds(g * LN, LN)] + 0.01)
        return partial + jnp.maximum(1.0 + xs * ap - an, 0.0)

    partial = lax.fori_loop(0, CGROUPS, group, zeros)
    part_v[...] = partial
    pltpu.sync_copy(part_v, out_hbm.at[wid])


@jax.jit
def _sc_call(table2, rel, rel_neg, pr, pr_path):
    mesh = plsc.VectorSubcoreMesh(core_axis_name="c", subcore_axis_name="s")
    kfn = pl.kernel(
        _sc_body,
        out_type=jax.ShapeDtypeStruct((NW, LN), jnp.float32),
        mesh=mesh,
        compiler_params=pltpu.CompilerParams(
            needs_layout_passes=False, use_tc_tiling_on_sc=False),
        scratch_types=[
            pltpu.VMEM((SLOTS,), jnp.int32),
            pltpu.VMEM((SLOTS,), jnp.int32),
            pltpu.VMEM((KCH, 128), jnp.int32),
            pltpu.VMEM((NIDX, LN), jnp.float32),
            pltpu.VMEM((BPW,), jnp.float32),
            pltpu.VMEM((BPW,), jnp.float32),
            pltpu.VMEM((20, LN), jnp.float32),
            pltpu.VMEM((GR, LN), jnp.float32),
            pltpu.VMEM((LN,), jnp.float32),
            pltpu.SemaphoreType.DMA,
        ],
    )
    return kfn(table2, rel, rel_neg, pr, pr_path)


def kernel(rel_table, paths, rel, rel_neg, pr, pr_path):
    del paths
    table2 = rel_table.reshape(-1, LN)
    part = _sc_call(table2, rel.astype(jnp.int32), rel_neg.astype(jnp.int32),
                    pr, pr_path)
    return jnp.sum(part)
